# Initial kernel scaffold; baseline (speedup 1.0000x reference)
import jax
import jax.numpy as jnp
from jax.experimental import pallas as pl


def kernel(f1, f2, f3, emb1, emb2, W, b):
    raise NotImplementedError("write your pallas kernel here")


# speedup vs baseline: 1.0813x; 1.0813x over previous
import functools

import jax
import jax.numpy as jnp
from jax import lax
from jax.experimental import pallas as pl
from jax.experimental.pallas import tpu as pltpu
from jax.experimental.pallas import tpu_sc as plsc

_B = 16384
_V = 1000000
_D = 16
_NC = 2
_NS = 16
_NW = _NC * _NS
_BPW = _B // _NW
_CHUNK = 128
_NCHUNK = _BPW // _CHUNK
_G = _BPW // _D

_mesh = plsc.VectorSubcoreMesh(core_axis_name="c", subcore_axis_name="s")


@functools.partial(
    pl.kernel,
    out_type=jax.ShapeDtypeStruct((_NW, _BPW), jnp.float32),
    mesh=_mesh,
    scratch_types=[
        pltpu.VMEM((_NCHUNK, _CHUNK), jnp.int32),
        pltpu.VMEM((_NCHUNK, _CHUNK), jnp.int32),
        pltpu.VMEM((_NCHUNK, _CHUNK), jnp.int32),
        pltpu.VMEM((_BPW, _D), jnp.float32),
        pltpu.VMEM((_BPW, _D), jnp.float32),
        pltpu.VMEM((_BPW, _D), jnp.float32),
        pltpu.VMEM((64,), jnp.float32),
        pltpu.VMEM((_BPW,), jnp.float32),
        pltpu.SemaphoreType.DMA,
    ],
    compiler_params=pltpu.CompilerParams(
        needs_layout_passes=False, use_tc_tiling_on_sc=False),
)
def _emb_dense_sc(f1_hbm, f2_hbm, f3_hbm, emb1_hbm, emb2_hbm, wb_hbm, out_hbm,
                  idx1, idx2, idx3, rows1, rows2, rows3, wv, out_v, sem):
    wid = lax.axis_index("s") * _NC + lax.axis_index("c")

    pltpu.sync_copy(f1_hbm.at[wid], idx1)
    pltpu.sync_copy(f2_hbm.at[wid], idx2)
    pltpu.sync_copy(f3_hbm.at[wid], idx3)
    pltpu.sync_copy(wb_hbm, wv)

    copies = []
    for j in range(_NCHUNK):
        sl = pl.ds(j * _CHUNK, _CHUNK)
        copies.append(pltpu.async_copy(emb1_hbm.at[idx1.at[j]], rows1.at[sl], sem))
        copies.append(pltpu.async_copy(emb1_hbm.at[idx2.at[j]], rows2.at[sl], sem))
        copies.append(pltpu.async_copy(emb2_hbm.at[idx3.at[j]], rows3.at[sl], sem))
    for cp in copies:
        cp.wait()

    bias = wv[pl.ds(48, _D)]
    w_vecs = (wv[pl.ds(0, _D)], wv[pl.ds(_D, _D)], wv[pl.ds(2 * _D, _D)])

    iota = lax.iota(jnp.int32, _D)

    def group_body(g, carry):
        rid = g * _D + iota
        accs = []
        for k, rows in enumerate((rows1, rows2, rows3)):
            wk = w_vecs[k]
            acc_e = jnp.zeros((_D,), jnp.float32)
            acc_o = jnp.zeros((_D,), jnp.float32)
            for d in range(0, _D, 2):
                col_e = plsc.load_gather(rows, [rid, jnp.full((_D,), d, jnp.int32)])
                col_o = plsc.load_gather(
                    rows, [rid, jnp.full((_D,), d + 1, jnp.int32)])
                acc_e = acc_e + col_e * wk[d]
                acc_o = acc_o + col_o * wk[d + 1]
            accs.append(acc_e + acc_o)
        out_v[pl.ds(g * _D, _D)] = bias + (accs[0] + accs[1]) + accs[2]
        return carry

    lax.fori_loop(0, _G, group_body, 0)

    pltpu.sync_copy(out_v, out_hbm.at[wid])


def kernel(f1, f2, f3, emb1, emb2, W, b):
    f1 = f1.astype(jnp.int32).reshape(_NW, _NCHUNK, _CHUNK)
    f2 = f2.astype(jnp.int32).reshape(_NW, _NCHUNK, _CHUNK)
    f3 = f3.astype(jnp.int32).reshape(_NW, _NCHUNK, _CHUNK)
    wb = jnp.concatenate([
        W.astype(jnp.float32).reshape(48),
        jnp.broadcast_to(b.astype(jnp.float32), (16,)),
    ])
    out = _emb_dense_sc(f1, f2, f3, emb1, emb2, wb)
    return out.reshape(_B, 1)


# speedup vs baseline: 6.9626x; 6.4392x over previous
import functools

import jax
import jax.numpy as jnp
from jax import lax
from jax.experimental import pallas as pl
from jax.experimental.pallas import tpu as pltpu
from jax.experimental.pallas import tpu_sc as plsc

_B = 16384
_V = 1000000
_D = 16
_NC = 2
_NS = 16
_NW = _NC * _NS
_BPW = _B // _NW
_CHUNK = 128
_NCHUNK = _BPW // _CHUNK
_G = _BPW // _D

_BC = 16384
_NBLK = -(-_V // _BC)

_mesh = plsc.VectorSubcoreMesh(core_axis_name="c", subcore_axis_name="s")


def _proj_body(e1_ref, e2_ref, wm_ref, b_ref, p_ref):
    e1 = e1_ref[...]
    e2 = e2_ref[...]
    wm = wm_ref[...]
    p01 = jax.lax.dot_general(
        wm[0:2], e1, (((1,), (0,)), ((), ())),
        preferred_element_type=jnp.float32)
    p2 = jax.lax.dot_general(
        wm[2:3], e2, (((1,), (0,)), ((), ())),
        preferred_element_type=jnp.float32)
    p_ref[...] = jnp.concatenate([p01, p2 + b_ref[0, 0]], axis=0)


_proj = pl.pallas_call(
    _proj_body,
    grid=(_NBLK,),
    in_specs=[
        pl.BlockSpec((_D, _BC), lambda i: (0, i)),
        pl.BlockSpec((_D, _BC), lambda i: (0, i)),
        pl.BlockSpec((3, _D), lambda i: (0, 0)),
        pl.BlockSpec((1, 1), lambda i: (0, 0), memory_space=pltpu.SMEM),
    ],
    out_specs=pl.BlockSpec((3, _BC), lambda i: (0, i)),
    out_shape=jax.ShapeDtypeStruct((3, _V), jnp.float32),
)


@functools.partial(
    pl.kernel,
    out_type=jax.ShapeDtypeStruct((_NW, _BPW), jnp.float32),
    mesh=_mesh,
    scratch_types=[
        pltpu.VMEM((_NCHUNK, _CHUNK), jnp.int32),
        pltpu.VMEM((_NCHUNK, _CHUNK), jnp.int32),
        pltpu.VMEM((_NCHUNK, _CHUNK), jnp.int32),
        pltpu.VMEM((_BPW,), jnp.float32),
        pltpu.VMEM((_BPW,), jnp.float32),
        pltpu.VMEM((_BPW,), jnp.float32),
        pltpu.VMEM((_BPW,), jnp.float32),
        pltpu.SemaphoreType.DMA,
    ],
    compiler_params=pltpu.CompilerParams(
        needs_layout_passes=False, use_tc_tiling_on_sc=False),
)
def _gather_add_sc(f1_hbm, f2_hbm, f3_hbm, p_hbm, out_hbm,
                   idx1, idx2, idx3, g1, g2, g3, out_v, sem):
    wid = lax.axis_index("s") * _NC + lax.axis_index("c")

    pltpu.sync_copy(f1_hbm.at[wid], idx1)
    pltpu.sync_copy(f2_hbm.at[wid], idx2)
    pltpu.sync_copy(f3_hbm.at[wid], idx3)

    copies = []
    for j in range(_NCHUNK):
        sl = pl.ds(j * _CHUNK, _CHUNK)
        copies.append(pltpu.async_copy(p_hbm.at[0].at[idx1.at[j]], g1.at[sl], sem))
        copies.append(pltpu.async_copy(p_hbm.at[1].at[idx2.at[j]], g2.at[sl], sem))
        copies.append(pltpu.async_copy(p_hbm.at[2].at[idx3.at[j]], g3.at[sl], sem))
    for cp in copies:
        cp.wait()

    def group_body(g, carry):
        sl = pl.ds(g * _D, _D)
        out_v[sl] = g1[sl] + g2[sl] + g3[sl]
        return carry

    lax.fori_loop(0, _G, group_body, 0)

    pltpu.sync_copy(out_v, out_hbm.at[wid])


def kernel(f1, f2, f3, emb1, emb2, W, b):
    f1 = f1.astype(jnp.int32).reshape(_NW, _NCHUNK, _CHUNK)
    f2 = f2.astype(jnp.int32).reshape(_NW, _NCHUNK, _CHUNK)
    f3 = f3.astype(jnp.int32).reshape(_NW, _NCHUNK, _CHUNK)
    wm = W.astype(jnp.float32).reshape(3, _D)
    bm = b.astype(jnp.float32).reshape(1, 1)
    p = _proj(emb1.T, emb2.T, wm, bm)
    out = _gather_add_sc(f1, f2, f3, p)
    return out.reshape(_B, 1)
